# baseline (speedup 1.0000x reference)
import functools

import jax
import jax.numpy as jnp
from jax import lax
from jax.experimental import pallas as pl
from jax.experimental.pallas import tpu as pltpu
from jax.experimental.pallas import tpu_sc as plsc

NUM_CORES = 2
NUM_SUBCORES = 16
LANES = 16
NW = NUM_CORES * NUM_SUBCORES
B = 16384
NE = 1000000
D = 64
BPW = B // NW

NBLK = 7813
FULLB = 7812
BASE_BLKS = FULLB // NW
EXTRA = FULLB - BASE_BLKS * NW
NHIT = 2064
NBKT = 16
RING = 8
NBUF = 8
BKTCAP = 272
SENT = 0x7FFFFFF0

CHUNK = 128
NCH = BPW // CHUNK
KD = D // LANES


def _rsqrt(x):
  i = lax.bitcast_convert_type(x, jnp.int32)
  i = jnp.int32(0x5F3759DF) - lax.shift_right_logical(i, 1)
  y = lax.bitcast_convert_type(i, jnp.float32)
  for _ in range(2):
    y = y * (1.5 - 0.5 * x * y * y)
  return y


def _scan_body(hidx_hbm, tidx_hbm, ent_hbm, scr_hbm,
               all_v, hitI, hitP, bktI, bktP, bhC, bhP, blk_ref, xrow,
               rows_v, semB, semW):
  wid = lax.axis_index("s") * NUM_CORES + lax.axis_index("c")
  blk0 = BASE_BLKS * wid + jnp.minimum(wid, EXTRA)
  nblk = jnp.where((wid < EXTRA) | (wid == NW - 1), BASE_BLKS + 1, BASE_BLKS)
  lo = blk0 * 128
  hi = (blk0 + nblk) * 128
  iota = lax.iota(jnp.int32, LANES)

  def fetch(b, par):
    start = pl.multiple_of((blk0 + b) * 128, 128)
    is_tail = (blk0 + b) == (NBLK - 1)

    @pl.when(jnp.logical_not(is_tail))
    def _():
      pltpu.async_copy(ent_hbm.at[:, :, :, pl.ds(start, 128)],
                       all_blk(par), semB)

    @pl.when(is_tail)
    def _():
      pltpu.async_copy(ent_hbm.at[:, :, :, pl.ds(start, 64)],
                       tail_blk(par), semB)

  def all_blk(par):
    return blk_ref.at[pl.ds(par, 1)]

  def tail_blk(par):
    return blk_ref.at[pl.ds(par, 1), :, :, pl.ds(0, 64)]

  def waitblk(b, par):
    is_tail = (blk0 + b) == (NBLK - 1)

    @pl.when(jnp.logical_not(is_tail))
    def _():
      pltpu.make_async_copy(ent_hbm.at[:, :, :, pl.ds(0, 128)],
                            all_blk(par), semB).wait()

    @pl.when(is_tail)
    def _():
      pltpu.make_async_copy(ent_hbm.at[:, :, :, pl.ds(0, 64)],
                            tail_blk(par), semB).wait()

  for i in range(NBUF):
    fetch(i, i)

  pltpu.sync_copy(hidx_hbm, all_v.at[pl.ds(0, B)])
  pltpu.sync_copy(tidx_hbm, all_v.at[pl.ds(B, B)])

  def fillA(q, c):
    hitI[pl.ds(q * LANES, LANES)] = jnp.full((LANES,), SENT, jnp.int32)
    return c
  lax.fori_loop(0, NHIT // LANES, fillA, 0)

  def fillB(q, c):
    bktI[pl.ds(q * LANES, LANES)] = jnp.full((LANES,), SENT, jnp.int32)
    return c
  lax.fori_loop(0, (NBKT * BKTCAP) // LANES, fillB, 0)

  def passA(vi, cursor):
    v = all_v[pl.ds(vi * LANES, LANES)]
    m = (v >= lo) & (v < hi)
    cnt = plsc.all_reduce_population_count(m)[0]
    plsc.store_compressed(hitI.at[pl.ds(cursor, LANES)], v, mask=m)
    plsc.store_compressed(hitP.at[pl.ds(cursor, LANES)],
                          vi * LANES + iota, mask=m)
    return cursor + cnt
  lax.fori_loop(0, (2 * B) // LANES, passA, jnp.int32(0))

  def passB(vi, curs):
    v = hitI[pl.ds(vi * LANES, LANES)]
    p = hitP[pl.ds(vi * LANES, LANES)]
    g = lax.shift_right_logical(lax.shift_right_logical(v, 7) - blk0, 4)
    out = []
    for gg in range(NBKT):
      m = g == gg
      cnt = plsc.all_reduce_population_count(m)[0]
      plsc.store_compressed(
          bktI.at[pl.ds(gg * BKTCAP + curs[gg], LANES)], v, mask=m)
      plsc.store_compressed(
          bktP.at[pl.ds(gg * BKTCAP + curs[gg], LANES)], p, mask=m)
      out.append(curs[gg] + cnt)
    return tuple(out)
  lax.fori_loop(0, NHIT // LANES, passB, (jnp.int32(0),) * NBKT)


  def block_step(b, carry):
    par = b & (NBUF - 1)
    waitblk(b, par)

    babs = blk0 + b
    g = lax.shift_right_logical(b, 4)

    def match(vi, mm):
      v = bktI[pl.ds(g * BKTCAP + vi * LANES, LANES)]
      p = bktP[pl.ds(g * BKTCAP + vi * LANES, LANES)]
      m = lax.shift_right_logical(v, 7) == babs
      cnt = plsc.all_reduce_population_count(m)[0]
      plsc.store_compressed(bhC.at[pl.ds(mm, LANES)], v & 127, mask=m)
      plsc.store_compressed(bhP.at[pl.ds(mm, LANES)], p, mask=m)
      return mm + cnt
    nhits = lax.fori_loop(0, BKTCAP // LANES, match, jnp.int32(0))

    def grp(q, gctr):
      slot = gctr & (RING - 1)
      slot_splat = slot + jnp.zeros((LANES,), jnp.int32)
      cnt = plsc.load_gather(rows_v, [slot_splat])[0]

      def drain(i, c3):
        pltpu.make_async_copy(scr_hbm.at[pl.ds(0, D)],
                              xrow.at[pl.ds(0, D)], semW).wait()
        return c3
      lax.fori_loop(0, cnt, drain, 0)

      valid = iota < (nhits - q * LANES)
      nrows = jnp.minimum(nhits - q * LANES, LANES)
      cvec = bhC[pl.ds(q * LANES, LANES)]
      pvec = bhP[pl.ds(q * LANES, LANES)]
      sbase = slot * (LANES * D)
      for d in range(D):
        vals = plsc.load_gather(
            blk_ref,
            [jnp.full((LANES,), par, jnp.int32),
             jnp.full((LANES,), d // 8, jnp.int32),
             jnp.full((LANES,), d % 8, jnp.int32), cvec], mask=valid)
        plsc.store_scatter(xrow, [sbase + iota * D + d], vals, mask=valid)
      for k in range(LANES):
        @pl.when(q * LANES + k < nhits)
        def _(k=k):
          pltpu.async_copy(xrow.at[pl.ds(sbase + k * D, D)],
                           scr_hbm.at[pl.ds(pvec[k] * D, D)], semW)
      plsc.store_scatter(rows_v, [slot_splat],
                         nrows + jnp.zeros((LANES,), jnp.int32),
                         mask=iota == 0)
      return gctr + 1

    gout = lax.fori_loop(0, (nhits + LANES - 1) // LANES, grp, carry)

    @pl.when(b + NBUF < nblk)
    def _():
      fetch(b + NBUF, par)

    return gout

  rows_v[pl.ds(0, LANES)] = jnp.zeros((LANES,), jnp.int32)
  lax.fori_loop(0, nblk, block_step, jnp.int32(0))

  cnts = rows_v[pl.ds(0, LANES)]
  for ss in range(RING):
    def drainf(i, c3):
      pltpu.make_async_copy(scr_hbm.at[pl.ds(0, D)],
                            xrow.at[pl.ds(0, D)], semW).wait()
      return c3
    lax.fori_loop(0, cnts[ss], drainf, 0)


def _make_scan():
  @functools.partial(
      pl.kernel,
      out_type=jax.ShapeDtypeStruct((2 * B * D,), jnp.float32),
      mesh=plsc.VectorSubcoreMesh(core_axis_name="c", subcore_axis_name="s"),
      compiler_params=pltpu.CompilerParams(
          needs_layout_passes=False, use_tc_tiling_on_sc=True),
      scratch_types=[
          pltpu.VMEM((2 * B,), jnp.int32),
          pltpu.VMEM((NHIT,), jnp.int32),
          pltpu.VMEM((NHIT,), jnp.int32),
          pltpu.VMEM((NBKT * BKTCAP,), jnp.int32),
          pltpu.VMEM((NBKT * BKTCAP,), jnp.int32),
          pltpu.VMEM((BKTCAP + LANES,), jnp.int32),
          pltpu.VMEM((BKTCAP + LANES,), jnp.int32),
          pltpu.VMEM((NBUF, 8, 8, 128), jnp.float32),
          pltpu.VMEM((RING * LANES * D,), jnp.float32),
          pltpu.VMEM((LANES,), jnp.int32),
          pltpu.SemaphoreType.DMA,
          pltpu.SemaphoreType.DMA,
      ],
  )
  def scan(hidx, tidx, ent3, scr, all_v, hitI, hitP, bktI, bktP, bhC, bhP,
           blk, xrow, rows_v, semB, semW):
    _scan_body(hidx, tidx, ent3, scr, all_v, hitI, hitP, bktI, bktP, bhC,
               bhP, blk, xrow, rows_v, semB, semW)
  return scan


_scan = _make_scan()


def _score_body(ridx_hbm, rel_hbm, scr_hbm, out_hbm,
                ri_v, h_v, t_v, r_v, out_v, sem):
  wid = lax.axis_index("s") * NUM_CORES + lax.axis_index("c")
  base = wid * BPW
  pltpu.sync_copy(ridx_hbm.at[wid], ri_v)
  lanes_iota = lax.iota(jnp.int32, LANES)

  for j in range(NCH):
    pltpu.sync_copy(scr_hbm.at[pl.ds((base + j * CHUNK) * D, CHUNK * D)], h_v)
    pltpu.sync_copy(
        scr_hbm.at[pl.ds((B + base + j * CHUNK) * D, CHUNK * D)], t_v)
    pltpu.async_copy(rel_hbm.at[ri_v.at[j]], r_v, sem).wait()

    def group(g, carry):
      svec = jnp.zeros((LANES,), jnp.float32)
      for k in range(LANES):
        slot = g * LANES + k

        def flatrow(ref):
          parts = []
          acc = None
          for kk in range(KD):
            v = ref[pl.ds(slot * D + kk * LANES, LANES)]
            parts.append(v)
            acc = v * v if acc is None else acc + v * v
          return parts, jnp.sum(acc)

        def relrow():
          parts = []
          acc = None
          for kk in range(KD):
            v = r_v[slot, pl.ds(kk * LANES, LANES)]
            parts.append(v)
            acc = v * v if acc is None else acc + v * v
          return parts, jnp.sum(acc)

        hp, sh = flatrow(h_v)
        tp, st = flatrow(t_v)
        rp, sr = relrow()
        ih = _rsqrt(jnp.maximum(sh, 1e-24))
        ir = _rsqrt(jnp.maximum(sr, 1e-24))
        it = _rsqrt(jnp.maximum(st, 1e-24))
        acc = None
        for kk in range(KD):
          term = jnp.abs(hp[kk] * ih + rp[kk] * ir - tp[kk] * it)
          acc = term if acc is None else acc + term
        svec = jnp.where(lanes_iota == k, jnp.sum(acc), svec)

      out_v[pl.ds(j * CHUNK + g * LANES, LANES)] = svec
      return carry

    lax.fori_loop(0, CHUNK // LANES, group, 0)

  pltpu.sync_copy(out_v, out_hbm.at[pl.ds(base, BPW)])


@functools.partial(
    pl.kernel,
    out_type=jax.ShapeDtypeStruct((B,), jnp.float32),
    mesh=plsc.VectorSubcoreMesh(core_axis_name="c", subcore_axis_name="s"),
    compiler_params=pltpu.CompilerParams(
        needs_layout_passes=False, use_tc_tiling_on_sc=True),
    scratch_types=[
        pltpu.VMEM((NCH, CHUNK), jnp.int32),
        pltpu.VMEM((CHUNK * D,), jnp.float32),
        pltpu.VMEM((CHUNK * D,), jnp.float32),
        pltpu.VMEM((CHUNK, 128), jnp.float32),
        pltpu.VMEM((BPW,), jnp.float32),
        pltpu.SemaphoreType.DMA,
    ],
)
def _score(ridx, rel_pad, scr, out, ri_v, h_v, t_v, r_v, out_v, sem):
  _score_body(ridx, rel_pad, scr, out, ri_v, h_v, t_v, r_v, out_v, sem)


def kernel(data, ent_emb, rel_emb):
  data = data.astype(jnp.int32)
  hidx = data[:, 0]
  ridx = data[:, 1].reshape(NW, NCH, CHUNK)
  tidx = data[:, 2]
  ent3 = ent_emb.T.reshape(1, 8, 8, NE)
  rel_pad = jnp.pad(rel_emb, ((0, 0), (0, 128 - D)))
  scr = _scan(hidx, tidx, ent3)
  return _score(ridx, rel_pad, scr)


# speedup vs baseline: 1.0460x; 1.0460x over previous
import functools

import jax
import jax.numpy as jnp
from jax import lax
from jax.experimental import pallas as pl
from jax.experimental.pallas import tpu as pltpu
from jax.experimental.pallas import tpu_sc as plsc

NUM_CORES = 2
NUM_SUBCORES = 16
LANES = 16
NW = NUM_CORES * NUM_SUBCORES
B = 16384
NE = 1000000
D = 64
BPW = B // NW

NBLK = 7813
FULLB = 7812
BASE_BLKS = FULLB // NW
EXTRA = FULLB - BASE_BLKS * NW
NHIT = 2064
NBKT = 16
RING = 8
NBUF = 8
BKTCAP = 272
SENT = 0x7FFFFFF0

CHUNK = 128
NCH = BPW // CHUNK
KD = D // LANES


def _rsqrt(x):
  i = lax.bitcast_convert_type(x, jnp.int32)
  i = jnp.int32(0x5F3759DF) - lax.shift_right_logical(i, 1)
  y = lax.bitcast_convert_type(i, jnp.float32)
  for _ in range(2):
    y = y * (1.5 - 0.5 * x * y * y)
  return y


def _scan_body(hidx_hbm, tidx_hbm, ent_hbm, scr_hbm,
               all_v, hitI, hitP, bktI, bktP, bhC, bhP, blk_ref, xrow,
               rows_v, semB, semW):
  wid = lax.axis_index("s") * NUM_CORES + lax.axis_index("c")
  blk0 = BASE_BLKS * wid + jnp.minimum(wid, EXTRA)
  nblk = jnp.where((wid < EXTRA) | (wid == NW - 1), BASE_BLKS + 1, BASE_BLKS)
  lo = blk0 * 128
  hi = (blk0 + nblk) * 128
  iota = lax.iota(jnp.int32, LANES)

  def fetch(b, par):
    start = pl.multiple_of((blk0 + b) * 128, 128)
    is_tail = (blk0 + b) == (NBLK - 1)

    @pl.when(jnp.logical_not(is_tail))
    def _():
      pltpu.async_copy(ent_hbm.at[:, :, :, pl.ds(start, 128)],
                       all_blk(par), semB)

    @pl.when(is_tail)
    def _():
      pltpu.async_copy(ent_hbm.at[:, :, :, pl.ds(start, 64)],
                       tail_blk(par), semB)

  def all_blk(par):
    return blk_ref.at[pl.ds(par, 1)]

  def tail_blk(par):
    return blk_ref.at[pl.ds(par, 1), :, :, pl.ds(0, 64)]

  def waitblk(b, par):
    is_tail = (blk0 + b) == (NBLK - 1)

    @pl.when(jnp.logical_not(is_tail))
    def _():
      pltpu.make_async_copy(ent_hbm.at[:, :, :, pl.ds(0, 128)],
                            all_blk(par), semB).wait()

    @pl.when(is_tail)
    def _():
      pltpu.make_async_copy(ent_hbm.at[:, :, :, pl.ds(0, 64)],
                            tail_blk(par), semB).wait()

  for i in range(NBUF):
    fetch(i, i)

  pltpu.sync_copy(hidx_hbm, all_v.at[pl.ds(0, B)])
  pltpu.sync_copy(tidx_hbm, all_v.at[pl.ds(B, B)])

  def fillA(q, c):
    hitI[pl.ds(q * LANES, LANES)] = jnp.full((LANES,), SENT, jnp.int32)
    return c
  lax.fori_loop(0, NHIT // LANES, fillA, 0)

  def fillB(q, c):
    bktI[pl.ds(q * LANES, LANES)] = jnp.full((LANES,), SENT, jnp.int32)
    return c
  lax.fori_loop(0, (NBKT * BKTCAP) // LANES, fillB, 0)

  def passA(vi, cursor):
    v = all_v[pl.ds(vi * LANES, LANES)]
    m = (v >= lo) & (v < hi)
    cnt = plsc.all_reduce_population_count(m)[0]
    plsc.store_compressed(hitI.at[pl.ds(cursor, LANES)], v, mask=m)
    plsc.store_compressed(hitP.at[pl.ds(cursor, LANES)],
                          vi * LANES + iota, mask=m)
    return cursor + cnt
  lax.fori_loop(0, (2 * B) // LANES, passA, jnp.int32(0))

  def passB(vi, curs):
    v = hitI[pl.ds(vi * LANES, LANES)]
    p = hitP[pl.ds(vi * LANES, LANES)]
    g = lax.shift_right_logical(lax.shift_right_logical(v, 7) - blk0, 4)
    out = []
    for gg in range(NBKT):
      m = g == gg
      cnt = plsc.all_reduce_population_count(m)[0]
      plsc.store_compressed(
          bktI.at[pl.ds(gg * BKTCAP + curs[gg], LANES)], v, mask=m)
      plsc.store_compressed(
          bktP.at[pl.ds(gg * BKTCAP + curs[gg], LANES)], p, mask=m)
      out.append(curs[gg] + cnt)
    return tuple(out)
  lax.fori_loop(0, NHIT // LANES, passB, (jnp.int32(0),) * NBKT)


  def block_step(b, carry):
    par = b & (NBUF - 1)
    waitblk(b, par)

    babs = blk0 + b
    g = lax.shift_right_logical(b, 4)

    def match(vi, mm):
      v = bktI[pl.ds(g * BKTCAP + vi * LANES, LANES)]
      p = bktP[pl.ds(g * BKTCAP + vi * LANES, LANES)]
      m = lax.shift_right_logical(v, 7) == babs
      cnt = plsc.all_reduce_population_count(m)[0]
      plsc.store_compressed(bhC.at[pl.ds(mm, LANES)], v & 127, mask=m)
      plsc.store_compressed(bhP.at[pl.ds(mm, LANES)], p, mask=m)
      return mm + cnt
    nhits = lax.fori_loop(0, BKTCAP // LANES, match, jnp.int32(0))

    def grp(q, gctr):
      slot = gctr & (RING - 1)
      slot_splat = slot + jnp.zeros((LANES,), jnp.int32)
      cnt = plsc.load_gather(rows_v, [slot_splat])[0]

      def drain(i, c3):
        pltpu.make_async_copy(scr_hbm.at[pl.ds(0, D)],
                              xrow.at[pl.ds(0, D)], semW).wait()
        return c3
      lax.fori_loop(0, cnt, drain, 0)

      valid = iota < (nhits - q * LANES)
      nrows = jnp.minimum(nhits - q * LANES, LANES)
      cvec = bhC[pl.ds(q * LANES, LANES)]
      pvec = bhP[pl.ds(q * LANES, LANES)]
      sbase = slot * (LANES * D)
      for d in range(D):
        vals = plsc.load_gather(
            blk_ref,
            [jnp.full((LANES,), par, jnp.int32),
             jnp.full((LANES,), d // 8, jnp.int32),
             jnp.full((LANES,), d % 8, jnp.int32), cvec], mask=valid)
        plsc.store_scatter(xrow, [sbase + iota * D + d], vals, mask=valid)
      for k in range(LANES):
        @pl.when(q * LANES + k < nhits)
        def _(k=k):
          pltpu.async_copy(xrow.at[pl.ds(sbase + k * D, D)],
                           scr_hbm.at[pl.ds(pvec[k] * D, D)], semW)
      plsc.store_scatter(rows_v, [slot_splat],
                         nrows + jnp.zeros((LANES,), jnp.int32),
                         mask=iota == 0)
      return gctr + 1

    gout = lax.fori_loop(0, (nhits + LANES - 1) // LANES, grp, carry)

    @pl.when(b + NBUF < nblk)
    def _():
      fetch(b + NBUF, par)

    return gout

  rows_v[pl.ds(0, LANES)] = jnp.zeros((LANES,), jnp.int32)
  lax.fori_loop(0, nblk, block_step, jnp.int32(0))

  cnts = rows_v[pl.ds(0, LANES)]
  for ss in range(RING):
    def drainf(i, c3):
      pltpu.make_async_copy(scr_hbm.at[pl.ds(0, D)],
                            xrow.at[pl.ds(0, D)], semW).wait()
      return c3
    lax.fori_loop(0, cnts[ss], drainf, 0)


def _make_scan():
  @functools.partial(
      pl.kernel,
      out_type=jax.ShapeDtypeStruct((2 * B * D,), jnp.float32),
      mesh=plsc.VectorSubcoreMesh(core_axis_name="c", subcore_axis_name="s"),
      compiler_params=pltpu.CompilerParams(
          needs_layout_passes=False, use_tc_tiling_on_sc=True),
      scratch_types=[
          pltpu.VMEM((2 * B,), jnp.int32),
          pltpu.VMEM((NHIT,), jnp.int32),
          pltpu.VMEM((NHIT,), jnp.int32),
          pltpu.VMEM((NBKT * BKTCAP,), jnp.int32),
          pltpu.VMEM((NBKT * BKTCAP,), jnp.int32),
          pltpu.VMEM((BKTCAP + LANES,), jnp.int32),
          pltpu.VMEM((BKTCAP + LANES,), jnp.int32),
          pltpu.VMEM((NBUF, 8, 8, 128), jnp.float32),
          pltpu.VMEM((RING * LANES * D,), jnp.float32),
          pltpu.VMEM((LANES,), jnp.int32),
          pltpu.SemaphoreType.DMA,
          pltpu.SemaphoreType.DMA,
      ],
  )
  def scan(hidx, tidx, ent3, scr, all_v, hitI, hitP, bktI, bktP, bhC, bhP,
           blk, xrow, rows_v, semB, semW):
    _scan_body(hidx, tidx, ent3, scr, all_v, hitI, hitP, bktI, bktP, bhC,
               bhP, blk, xrow, rows_v, semB, semW)
  return scan


_scan = _make_scan()


def _score_body(ridx_hbm, rel_hbm, scr_hbm, out_hbm,
                ri_v, h_v, t_v, r_v, out_v, sem):
  wid = lax.axis_index("s") * NUM_CORES + lax.axis_index("c")
  base = wid * BPW
  pltpu.sync_copy(ridx_hbm.at[wid], ri_v)
  lanes_iota = lax.iota(jnp.int32, LANES)
  HB = CHUNK * D

  def issue(j, par):
    pltpu.async_copy(scr_hbm.at[pl.ds((base + j * CHUNK) * D, HB)],
                     h_v.at[pl.ds(par * HB, HB)], sem)
    pltpu.async_copy(scr_hbm.at[pl.ds((B + base + j * CHUNK) * D, HB)],
                     t_v.at[pl.ds(par * HB, HB)], sem)
    pltpu.async_copy(rel_hbm.at[ri_v.at[j]],
                     r_v.at[pl.ds(par * CHUNK, CHUNK)], sem)

  def drain(par):
    pltpu.make_async_copy(scr_hbm.at[pl.ds(0, HB)],
                          h_v.at[pl.ds(par * HB, HB)], sem).wait()
    pltpu.make_async_copy(scr_hbm.at[pl.ds(0, HB)],
                          t_v.at[pl.ds(par * HB, HB)], sem).wait()
    pltpu.make_async_copy(rel_hbm.at[pl.ds(0, CHUNK)],
                          r_v.at[pl.ds(par * CHUNK, CHUNK)], sem).wait()

  issue(0, 0)
  for j in range(NCH):
    par = j & 1
    drain(par)
    if j + 1 < NCH:
      issue(j + 1, (j + 1) & 1)

    def group(g, carry):
      svec = jnp.zeros((LANES,), jnp.float32)
      for k in range(LANES):
        slot = g * LANES + k

        def flatrow(ref):
          parts = []
          acc = None
          for kk in range(KD):
            v = ref[pl.ds(par * HB + slot * D + kk * LANES, LANES)]
            parts.append(v)
            acc = v * v if acc is None else acc + v * v
          return parts, jnp.sum(acc)

        def relrow():
          parts = []
          acc = None
          for kk in range(KD):
            v = r_v[par * CHUNK + slot, pl.ds(kk * LANES, LANES)]
            parts.append(v)
            acc = v * v if acc is None else acc + v * v
          return parts, jnp.sum(acc)

        hp, sh = flatrow(h_v)
        tp, st = flatrow(t_v)
        rp, sr = relrow()
        ih = _rsqrt(jnp.maximum(sh, 1e-24))
        ir = _rsqrt(jnp.maximum(sr, 1e-24))
        it = _rsqrt(jnp.maximum(st, 1e-24))
        acc = None
        for kk in range(KD):
          term = jnp.abs(hp[kk] * ih + rp[kk] * ir - tp[kk] * it)
          acc = term if acc is None else acc + term
        svec = jnp.where(lanes_iota == k, jnp.sum(acc), svec)

      out_v[pl.ds(j * CHUNK + g * LANES, LANES)] = svec
      return carry

    lax.fori_loop(0, CHUNK // LANES, group, 0)

  pltpu.sync_copy(out_v, out_hbm.at[pl.ds(base, BPW)])


@functools.partial(
    pl.kernel,
    out_type=jax.ShapeDtypeStruct((B,), jnp.float32),
    mesh=plsc.VectorSubcoreMesh(core_axis_name="c", subcore_axis_name="s"),
    compiler_params=pltpu.CompilerParams(
        needs_layout_passes=False, use_tc_tiling_on_sc=True),
    scratch_types=[
        pltpu.VMEM((NCH, CHUNK), jnp.int32),
        pltpu.VMEM((2 * CHUNK * D,), jnp.float32),
        pltpu.VMEM((2 * CHUNK * D,), jnp.float32),
        pltpu.VMEM((2 * CHUNK, 128), jnp.float32),
        pltpu.VMEM((BPW,), jnp.float32),
        pltpu.SemaphoreType.DMA,
    ],
)
def _score(ridx, rel_pad, scr, out, ri_v, h_v, t_v, r_v, out_v, sem):
  _score_body(ridx, rel_pad, scr, out, ri_v, h_v, t_v, r_v, out_v, sem)


def kernel(data, ent_emb, rel_emb):
  data = data.astype(jnp.int32)
  hidx = data[:, 0]
  ridx = data[:, 1].reshape(NW, NCH, CHUNK)
  tidx = data[:, 2]
  ent3 = ent_emb.T.reshape(1, 8, 8, NE)
  rel_pad = jnp.pad(rel_emb, ((0, 0), (0, 128 - D)))
  scr = _scan(hidx, tidx, ent3)
  return _score(ridx, rel_pad, scr)


# speedup vs baseline: 1.1596x; 1.1087x over previous
import functools

import jax
import jax.numpy as jnp
from jax import lax
from jax.experimental import pallas as pl
from jax.experimental.pallas import tpu as pltpu
from jax.experimental.pallas import tpu_sc as plsc

NUM_CORES = 2
NUM_SUBCORES = 16
LANES = 16
NW = NUM_CORES * NUM_SUBCORES
B = 16384
NE = 1000000
D = 64
BPW = B // NW

NBLK = 7813
FULLB = 7812
BASE_BLKS = FULLB // NW
EXTRA = FULLB - BASE_BLKS * NW
NHIT = 1360
NBKT = 16
RING = 8
NBUF = 8
BKTCAP = 160
SENT = 0x7FFFFFF0

CHUNK = 128
NCH = BPW // CHUNK
KD = D // LANES


def _rsqrt(x):
  i = lax.bitcast_convert_type(x, jnp.int32)
  i = jnp.int32(0x5F3759DF) - lax.shift_right_logical(i, 1)
  y = lax.bitcast_convert_type(i, jnp.float32)
  for _ in range(2):
    y = y * (1.5 - 0.5 * x * y * y)
  return y


def _scan_body(hidx_hbm, tidx_hbm, ent_hbm, scr_hbm,
               all_v, hitI, hitP, bktI, bktP, bhC, bhP, blk_ref, xrow,
               rows_v, semB, semW):
  wid = lax.axis_index("s") * NUM_CORES + lax.axis_index("c")
  blk0 = BASE_BLKS * wid + jnp.minimum(wid, EXTRA)
  nblk = jnp.where((wid < EXTRA) | (wid == NW - 1), BASE_BLKS + 1, BASE_BLKS)
  lo = blk0 * 128
  hi = (blk0 + nblk) * 128
  iota = lax.iota(jnp.int32, LANES)

  def fetch(b, par):
    start = pl.multiple_of((blk0 + b) * 128, 128)
    is_tail = (blk0 + b) == (NBLK - 1)

    @pl.when(jnp.logical_not(is_tail))
    def _():
      pltpu.async_copy(ent_hbm.at[:, :, :, pl.ds(start, 128)],
                       all_blk(par), semB)

    @pl.when(is_tail)
    def _():
      pltpu.async_copy(ent_hbm.at[:, :, :, pl.ds(start, 64)],
                       tail_blk(par), semB)

  def all_blk(par):
    return blk_ref.at[pl.ds(par, 1)]

  def tail_blk(par):
    return blk_ref.at[pl.ds(par, 1), :, :, pl.ds(0, 64)]

  def waitblk(b, par):
    is_tail = (blk0 + b) == (NBLK - 1)

    @pl.when(jnp.logical_not(is_tail))
    def _():
      pltpu.make_async_copy(ent_hbm.at[:, :, :, pl.ds(0, 128)],
                            all_blk(par), semB).wait()

    @pl.when(is_tail)
    def _():
      pltpu.make_async_copy(ent_hbm.at[:, :, :, pl.ds(0, 64)],
                            tail_blk(par), semB).wait()

  for i in range(NBUF):
    fetch(i, i)

  pltpu.sync_copy(hidx_hbm, all_v.at[pl.ds(0, B)])
  pltpu.sync_copy(tidx_hbm, all_v.at[pl.ds(B, B)])

  def fillA(q, c):
    hitI[pl.ds(q * LANES, LANES)] = jnp.full((LANES,), SENT, jnp.int32)
    return c
  lax.fori_loop(0, NHIT // LANES, fillA, 0)

  def fillB(q, c):
    bktI[pl.ds(q * LANES, LANES)] = jnp.full((LANES,), SENT, jnp.int32)
    return c
  lax.fori_loop(0, (NBKT * BKTCAP) // LANES, fillB, 0)

  def passA(vi, cursor):
    v = all_v[pl.ds(vi * LANES, LANES)]
    m = (v >= lo) & (v < hi)
    cnt = plsc.all_reduce_population_count(m)[0]
    plsc.store_compressed(hitI.at[pl.ds(cursor, LANES)], v, mask=m)
    plsc.store_compressed(hitP.at[pl.ds(cursor, LANES)],
                          vi * LANES + iota, mask=m)
    return cursor + cnt
  lax.fori_loop(0, (2 * B) // LANES, passA, jnp.int32(0))

  def passB(vi, curs):
    v = hitI[pl.ds(vi * LANES, LANES)]
    p = hitP[pl.ds(vi * LANES, LANES)]
    g = lax.shift_right_logical(lax.shift_right_logical(v, 7) - blk0, 4)
    out = []
    for gg in range(NBKT):
      m = g == gg
      cnt = plsc.all_reduce_population_count(m)[0]
      plsc.store_compressed(
          bktI.at[pl.ds(gg * BKTCAP + curs[gg], LANES)], v, mask=m)
      plsc.store_compressed(
          bktP.at[pl.ds(gg * BKTCAP + curs[gg], LANES)], p, mask=m)
      out.append(curs[gg] + cnt)
    return tuple(out)
  lax.fori_loop(0, NHIT // LANES, passB, (jnp.int32(0),) * NBKT)


  def block_step(b, carry):
    par = b & (NBUF - 1)
    waitblk(b, par)

    babs = blk0 + b
    g = lax.shift_right_logical(b, 4)

    def match(vi, mm):
      v = bktI[pl.ds(g * BKTCAP + vi * LANES, LANES)]
      p = bktP[pl.ds(g * BKTCAP + vi * LANES, LANES)]
      m = lax.shift_right_logical(v, 7) == babs
      cnt = plsc.all_reduce_population_count(m)[0]
      plsc.store_compressed(bhC.at[pl.ds(mm, LANES)], v & 127, mask=m)
      plsc.store_compressed(bhP.at[pl.ds(mm, LANES)], p, mask=m)
      return mm + cnt
    nhits = lax.fori_loop(0, BKTCAP // LANES, match, jnp.int32(0))

    def grp(q, gctr):
      slot = gctr & (RING - 1)
      slot_splat = slot + jnp.zeros((LANES,), jnp.int32)
      cnt = plsc.load_gather(rows_v, [slot_splat])[0]

      def drain(i, c3):
        pltpu.make_async_copy(scr_hbm.at[pl.ds(0, D)],
                              xrow.at[pl.ds(0, D)], semW).wait()
        return c3
      lax.fori_loop(0, cnt, drain, 0)

      valid = iota < (nhits - q * LANES)
      nrows = jnp.minimum(nhits - q * LANES, LANES)
      cvec = bhC[pl.ds(q * LANES, LANES)]
      pvec = bhP[pl.ds(q * LANES, LANES)]
      sbase = slot * (LANES * D)
      for d in range(D):
        vals = plsc.load_gather(
            blk_ref,
            [jnp.full((LANES,), par, jnp.int32),
             jnp.full((LANES,), d // 8, jnp.int32),
             jnp.full((LANES,), d % 8, jnp.int32), cvec], mask=valid)
        plsc.store_scatter(xrow, [sbase + iota * D + d], vals, mask=valid)
      for k in range(LANES):
        @pl.when(q * LANES + k < nhits)
        def _(k=k):
          pltpu.async_copy(xrow.at[pl.ds(sbase + k * D, D)],
                           scr_hbm.at[pl.ds(pvec[k] * D, D)], semW)
      plsc.store_scatter(rows_v, [slot_splat],
                         nrows + jnp.zeros((LANES,), jnp.int32),
                         mask=iota == 0)
      return gctr + 1

    gout = lax.fori_loop(0, (nhits + LANES - 1) // LANES, grp, carry)

    @pl.when(b + NBUF < nblk)
    def _():
      fetch(b + NBUF, par)

    return gout

  rows_v[pl.ds(0, LANES)] = jnp.zeros((LANES,), jnp.int32)
  lax.fori_loop(0, nblk, block_step, jnp.int32(0))

  cnts = rows_v[pl.ds(0, LANES)]
  for ss in range(RING):
    def drainf(i, c3):
      pltpu.make_async_copy(scr_hbm.at[pl.ds(0, D)],
                            xrow.at[pl.ds(0, D)], semW).wait()
      return c3
    lax.fori_loop(0, cnts[ss], drainf, 0)


def _make_scan():
  @functools.partial(
      pl.kernel,
      out_type=jax.ShapeDtypeStruct((2 * B * D,), jnp.float32),
      mesh=plsc.VectorSubcoreMesh(core_axis_name="c", subcore_axis_name="s"),
      compiler_params=pltpu.CompilerParams(
          needs_layout_passes=False, use_tc_tiling_on_sc=True),
      scratch_types=[
          pltpu.VMEM((2 * B,), jnp.int32),
          pltpu.VMEM((NHIT,), jnp.int32),
          pltpu.VMEM((NHIT,), jnp.int32),
          pltpu.VMEM((NBKT * BKTCAP,), jnp.int32),
          pltpu.VMEM((NBKT * BKTCAP,), jnp.int32),
          pltpu.VMEM((BKTCAP + LANES,), jnp.int32),
          pltpu.VMEM((BKTCAP + LANES,), jnp.int32),
          pltpu.VMEM((NBUF, 8, 8, 128), jnp.float32),
          pltpu.VMEM((RING * LANES * D,), jnp.float32),
          pltpu.VMEM((LANES,), jnp.int32),
          pltpu.SemaphoreType.DMA,
          pltpu.SemaphoreType.DMA,
      ],
  )
  def scan(hidx, tidx, ent3, scr, all_v, hitI, hitP, bktI, bktP, bhC, bhP,
           blk, xrow, rows_v, semB, semW):
    _scan_body(hidx, tidx, ent3, scr, all_v, hitI, hitP, bktI, bktP, bhC,
               bhP, blk, xrow, rows_v, semB, semW)
  return scan


_scan = _make_scan()


def _score_body(ridx_hbm, rel_hbm, scr_hbm, out_hbm,
                ri_v, h_v, t_v, r_v, out_v, sem):
  wid = lax.axis_index("s") * NUM_CORES + lax.axis_index("c")
  base = wid * BPW
  pltpu.sync_copy(ridx_hbm.at[wid], ri_v)
  lanes_iota = lax.iota(jnp.int32, LANES)
  HB = CHUNK * D

  def issue(j, par):
    pltpu.async_copy(scr_hbm.at[pl.ds((base + j * CHUNK) * D, HB)],
                     h_v.at[pl.ds(par * HB, HB)], sem)
    pltpu.async_copy(scr_hbm.at[pl.ds((B + base + j * CHUNK) * D, HB)],
                     t_v.at[pl.ds(par * HB, HB)], sem)
    pltpu.async_copy(rel_hbm.at[ri_v.at[j]],
                     r_v.at[pl.ds(par * CHUNK, CHUNK)], sem)

  def drain(par):
    pltpu.make_async_copy(scr_hbm.at[pl.ds(0, HB)],
                          h_v.at[pl.ds(par * HB, HB)], sem).wait()
    pltpu.make_async_copy(scr_hbm.at[pl.ds(0, HB)],
                          t_v.at[pl.ds(par * HB, HB)], sem).wait()
    pltpu.make_async_copy(rel_hbm.at[pl.ds(0, CHUNK)],
                          r_v.at[pl.ds(par * CHUNK, CHUNK)], sem).wait()

  issue(0, 0)
  for j in range(NCH):
    par = j & 1
    drain(par)
    if j + 1 < NCH:
      issue(j + 1, (j + 1) & 1)

    def group(g, carry):
      svec = jnp.zeros((LANES,), jnp.float32)
      for k in range(LANES):
        slot = g * LANES + k

        def flatrow(ref):
          parts = []
          acc = None
          for kk in range(KD):
            v = ref[pl.ds(par * HB + slot * D + kk * LANES, LANES)]
            parts.append(v)
            acc = v * v if acc is None else acc + v * v
          return parts, jnp.sum(acc)

        def relrow():
          parts = []
          acc = None
          for kk in range(KD):
            v = r_v[par * CHUNK + slot, pl.ds(kk * LANES, LANES)]
            parts.append(v)
            acc = v * v if acc is None else acc + v * v
          return parts, jnp.sum(acc)

        hp, sh = flatrow(h_v)
        tp, st = flatrow(t_v)
        rp, sr = relrow()
        ih = _rsqrt(jnp.maximum(sh, 1e-24))
        ir = _rsqrt(jnp.maximum(sr, 1e-24))
        it = _rsqrt(jnp.maximum(st, 1e-24))
        acc = None
        for kk in range(KD):
          term = jnp.abs(hp[kk] * ih + rp[kk] * ir - tp[kk] * it)
          acc = term if acc is None else acc + term
        svec = jnp.where(lanes_iota == k, jnp.sum(acc), svec)

      out_v[pl.ds(j * CHUNK + g * LANES, LANES)] = svec
      return carry

    lax.fori_loop(0, CHUNK // LANES, group, 0)

  pltpu.sync_copy(out_v, out_hbm.at[pl.ds(base, BPW)])


@functools.partial(
    pl.kernel,
    out_type=jax.ShapeDtypeStruct((B,), jnp.float32),
    mesh=plsc.VectorSubcoreMesh(core_axis_name="c", subcore_axis_name="s"),
    compiler_params=pltpu.CompilerParams(
        needs_layout_passes=False, use_tc_tiling_on_sc=True),
    scratch_types=[
        pltpu.VMEM((NCH, CHUNK), jnp.int32),
        pltpu.VMEM((2 * CHUNK * D,), jnp.float32),
        pltpu.VMEM((2 * CHUNK * D,), jnp.float32),
        pltpu.VMEM((2 * CHUNK, 128), jnp.float32),
        pltpu.VMEM((BPW,), jnp.float32),
        pltpu.SemaphoreType.DMA,
    ],
)
def _score(ridx, rel_pad, scr, out, ri_v, h_v, t_v, r_v, out_v, sem):
  _score_body(ridx, rel_pad, scr, out, ri_v, h_v, t_v, r_v, out_v, sem)


def kernel(data, ent_emb, rel_emb):
  data = data.astype(jnp.int32)
  hidx = data[:, 0]
  ridx = data[:, 1].reshape(NW, NCH, CHUNK)
  tidx = data[:, 2]
  ent3 = ent_emb.T.reshape(1, 8, 8, NE)
  rel_pad = jnp.pad(rel_emb, ((0, 0), (0, 128 - D)))
  scr = _scan(hidx, tidx, ent3)
  return _score(ridx, rel_pad, scr)


# speedup vs baseline: 1.1631x; 1.0030x over previous
import functools

import jax
import jax.numpy as jnp
from jax import lax
from jax.experimental import pallas as pl
from jax.experimental.pallas import tpu as pltpu
from jax.experimental.pallas import tpu_sc as plsc

NUM_CORES = 2
NUM_SUBCORES = 16
LANES = 16
NW = NUM_CORES * NUM_SUBCORES
B = 16384
NE = 1000000
D = 64
BPW = B // NW

NBLK = 7813
FULLB = 7812
BASE_BLKS = FULLB // NW
EXTRA = FULLB - BASE_BLKS * NW
NHIT = 1360
NBKT = 16
RING = 8
NBUF = 8
BKTCAP = 160
SENT = 0x7FFFFFF0

CHUNK = 128
NCH = BPW // CHUNK
KD = D // LANES


def _rsqrt(x):
  i = lax.bitcast_convert_type(x, jnp.int32)
  i = jnp.int32(0x5F3759DF) - lax.shift_right_logical(i, 1)
  y = lax.bitcast_convert_type(i, jnp.float32)
  for _ in range(2):
    y = y * (1.5 - 0.5 * x * y * y)
  return y


def _scan_body(hidx_hbm, tidx_hbm, ent_hbm, scr_hbm,
               all_v, hitI, hitP, bktI, bktP, bhC, bhP, blk_ref, xrow,
               rows_v, semB, semW):
  wid = lax.axis_index("s") * NUM_CORES + lax.axis_index("c")
  blk0 = BASE_BLKS * wid + jnp.minimum(wid, EXTRA)
  nblk = jnp.where((wid < EXTRA) | (wid == NW - 1), BASE_BLKS + 1, BASE_BLKS)
  lo = blk0 * 128
  hi = (blk0 + nblk) * 128
  iota = lax.iota(jnp.int32, LANES)

  def fetch(b, par):
    start = pl.multiple_of((blk0 + b) * 128, 128)
    is_tail = (blk0 + b) == (NBLK - 1)

    @pl.when(jnp.logical_not(is_tail))
    def _():
      pltpu.async_copy(ent_hbm.at[:, :, :, pl.ds(start, 128)],
                       all_blk(par), semB)

    @pl.when(is_tail)
    def _():
      pltpu.async_copy(ent_hbm.at[:, :, :, pl.ds(start, 64)],
                       tail_blk(par), semB)

  def all_blk(par):
    return blk_ref.at[pl.ds(par, 1)]

  def tail_blk(par):
    return blk_ref.at[pl.ds(par, 1), :, :, pl.ds(0, 64)]

  def waitblk(b, par):
    is_tail = (blk0 + b) == (NBLK - 1)

    @pl.when(jnp.logical_not(is_tail))
    def _():
      pltpu.make_async_copy(ent_hbm.at[:, :, :, pl.ds(0, 128)],
                            all_blk(par), semB).wait()

    @pl.when(is_tail)
    def _():
      pltpu.make_async_copy(ent_hbm.at[:, :, :, pl.ds(0, 64)],
                            tail_blk(par), semB).wait()

  for i in range(NBUF):
    fetch(i, i)

  pltpu.sync_copy(hidx_hbm, all_v.at[pl.ds(0, B)])
  pltpu.sync_copy(tidx_hbm, all_v.at[pl.ds(B, B)])

  def fillA(q, c):
    hitI[pl.ds(q * LANES, LANES)] = jnp.full((LANES,), SENT, jnp.int32)
    return c
  lax.fori_loop(0, NHIT // LANES, fillA, 0)

  def fillB(q, c):
    bktI[pl.ds(q * LANES, LANES)] = jnp.full((LANES,), SENT, jnp.int32)
    return c
  lax.fori_loop(0, (NBKT * BKTCAP) // LANES, fillB, 0)

  def passA(vi, cursor):
    v = all_v[pl.ds(vi * LANES, LANES)]
    m = (v >= lo) & (v < hi)
    cnt = plsc.all_reduce_population_count(m)[0]
    plsc.store_compressed(hitI.at[pl.ds(cursor, LANES)], v, mask=m)
    plsc.store_compressed(hitP.at[pl.ds(cursor, LANES)],
                          vi * LANES + iota, mask=m)
    return cursor + cnt
  lax.fori_loop(0, (2 * B) // LANES, passA, jnp.int32(0), unroll=2)

  def passB(vi, curs):
    v = hitI[pl.ds(vi * LANES, LANES)]
    p = hitP[pl.ds(vi * LANES, LANES)]
    g = lax.shift_right_logical(lax.shift_right_logical(v, 7) - blk0, 4)
    out = []
    for gg in range(NBKT):
      m = g == gg
      cnt = plsc.all_reduce_population_count(m)[0]
      plsc.store_compressed(
          bktI.at[pl.ds(gg * BKTCAP + curs[gg], LANES)], v, mask=m)
      plsc.store_compressed(
          bktP.at[pl.ds(gg * BKTCAP + curs[gg], LANES)], p, mask=m)
      out.append(curs[gg] + cnt)
    return tuple(out)
  lax.fori_loop(0, NHIT // LANES, passB, (jnp.int32(0),) * NBKT,
                unroll=2)


  def block_step(b, carry):
    par = b & (NBUF - 1)
    waitblk(b, par)

    babs = blk0 + b
    g = lax.shift_right_logical(b, 4)

    def match(vi, mm):
      v = bktI[pl.ds(g * BKTCAP + vi * LANES, LANES)]
      p = bktP[pl.ds(g * BKTCAP + vi * LANES, LANES)]
      m = lax.shift_right_logical(v, 7) == babs
      cnt = plsc.all_reduce_population_count(m)[0]
      plsc.store_compressed(bhC.at[pl.ds(mm, LANES)], v & 127, mask=m)
      plsc.store_compressed(bhP.at[pl.ds(mm, LANES)], p, mask=m)
      return mm + cnt
    nhits = lax.fori_loop(0, BKTCAP // LANES, match, jnp.int32(0))

    def grp(q, gctr):
      slot = gctr & (RING - 1)
      slot_splat = slot + jnp.zeros((LANES,), jnp.int32)
      cnt = plsc.load_gather(rows_v, [slot_splat])[0]

      def drain(i, c3):
        pltpu.make_async_copy(scr_hbm.at[pl.ds(0, D)],
                              xrow.at[pl.ds(0, D)], semW).wait()
        return c3
      lax.fori_loop(0, cnt, drain, 0)

      valid = iota < (nhits - q * LANES)
      nrows = jnp.minimum(nhits - q * LANES, LANES)
      cvec = bhC[pl.ds(q * LANES, LANES)]
      pvec = bhP[pl.ds(q * LANES, LANES)]
      sbase = slot * (LANES * D)
      for d in range(D):
        vals = plsc.load_gather(
            blk_ref,
            [jnp.full((LANES,), par, jnp.int32),
             jnp.full((LANES,), d // 8, jnp.int32),
             jnp.full((LANES,), d % 8, jnp.int32), cvec], mask=valid)
        plsc.store_scatter(xrow, [sbase + iota * D + d], vals, mask=valid)
      for k in range(LANES):
        @pl.when(q * LANES + k < nhits)
        def _(k=k):
          pltpu.async_copy(xrow.at[pl.ds(sbase + k * D, D)],
                           scr_hbm.at[pl.ds(pvec[k] * D, D)], semW)
      plsc.store_scatter(rows_v, [slot_splat],
                         nrows + jnp.zeros((LANES,), jnp.int32),
                         mask=iota == 0)
      return gctr + 1

    gout = lax.fori_loop(0, (nhits + LANES - 1) // LANES, grp, carry)

    @pl.when(b + NBUF < nblk)
    def _():
      fetch(b + NBUF, par)

    return gout

  rows_v[pl.ds(0, LANES)] = jnp.zeros((LANES,), jnp.int32)
  lax.fori_loop(0, nblk, block_step, jnp.int32(0))

  cnts = rows_v[pl.ds(0, LANES)]
  for ss in range(RING):
    def drainf(i, c3):
      pltpu.make_async_copy(scr_hbm.at[pl.ds(0, D)],
                            xrow.at[pl.ds(0, D)], semW).wait()
      return c3
    lax.fori_loop(0, cnts[ss], drainf, 0)


def _make_scan():
  @functools.partial(
      pl.kernel,
      out_type=jax.ShapeDtypeStruct((2 * B * D,), jnp.float32),
      mesh=plsc.VectorSubcoreMesh(core_axis_name="c", subcore_axis_name="s"),
      compiler_params=pltpu.CompilerParams(
          needs_layout_passes=False, use_tc_tiling_on_sc=True),
      scratch_types=[
          pltpu.VMEM((2 * B,), jnp.int32),
          pltpu.VMEM((NHIT,), jnp.int32),
          pltpu.VMEM((NHIT,), jnp.int32),
          pltpu.VMEM((NBKT * BKTCAP,), jnp.int32),
          pltpu.VMEM((NBKT * BKTCAP,), jnp.int32),
          pltpu.VMEM((BKTCAP + LANES,), jnp.int32),
          pltpu.VMEM((BKTCAP + LANES,), jnp.int32),
          pltpu.VMEM((NBUF, 8, 8, 128), jnp.float32),
          pltpu.VMEM((RING * LANES * D,), jnp.float32),
          pltpu.VMEM((LANES,), jnp.int32),
          pltpu.SemaphoreType.DMA,
          pltpu.SemaphoreType.DMA,
      ],
  )
  def scan(hidx, tidx, ent3, scr, all_v, hitI, hitP, bktI, bktP, bhC, bhP,
           blk, xrow, rows_v, semB, semW):
    _scan_body(hidx, tidx, ent3, scr, all_v, hitI, hitP, bktI, bktP, bhC,
               bhP, blk, xrow, rows_v, semB, semW)
  return scan


_scan = _make_scan()


def _score_body(ridx_hbm, rel_hbm, scr_hbm, out_hbm,
                ri_v, h_v, t_v, r_v, out_v, sem):
  wid = lax.axis_index("s") * NUM_CORES + lax.axis_index("c")
  base = wid * BPW
  pltpu.sync_copy(ridx_hbm.at[wid], ri_v)
  lanes_iota = lax.iota(jnp.int32, LANES)
  HB = CHUNK * D

  def issue(j, par):
    pltpu.async_copy(scr_hbm.at[pl.ds((base + j * CHUNK) * D, HB)],
                     h_v.at[pl.ds(par * HB, HB)], sem)
    pltpu.async_copy(scr_hbm.at[pl.ds((B + base + j * CHUNK) * D, HB)],
                     t_v.at[pl.ds(par * HB, HB)], sem)
    pltpu.async_copy(rel_hbm.at[ri_v.at[j]],
                     r_v.at[pl.ds(par * CHUNK, CHUNK)], sem)

  def drain(par):
    pltpu.make_async_copy(scr_hbm.at[pl.ds(0, HB)],
                          h_v.at[pl.ds(par * HB, HB)], sem).wait()
    pltpu.make_async_copy(scr_hbm.at[pl.ds(0, HB)],
                          t_v.at[pl.ds(par * HB, HB)], sem).wait()
    pltpu.make_async_copy(rel_hbm.at[pl.ds(0, CHUNK)],
                          r_v.at[pl.ds(par * CHUNK, CHUNK)], sem).wait()

  issue(0, 0)
  for j in range(NCH):
    par = j & 1
    drain(par)
    if j + 1 < NCH:
      issue(j + 1, (j + 1) & 1)

    def group(g, carry):
      svec = jnp.zeros((LANES,), jnp.float32)
      for k in range(LANES):
        slot = g * LANES + k

        def flatrow(ref):
          parts = []
          acc = None
          for kk in range(KD):
            v = ref[pl.ds(par * HB + slot * D + kk * LANES, LANES)]
            parts.append(v)
            acc = v * v if acc is None else acc + v * v
          return parts, jnp.sum(acc)

        def relrow():
          parts = []
          acc = None
          for kk in range(KD):
            v = r_v[par * CHUNK + slot, pl.ds(kk * LANES, LANES)]
            parts.append(v)
            acc = v * v if acc is None else acc + v * v
          return parts, jnp.sum(acc)

        hp, sh = flatrow(h_v)
        tp, st = flatrow(t_v)
        rp, sr = relrow()
        ih = _rsqrt(jnp.maximum(sh, 1e-24))
        ir = _rsqrt(jnp.maximum(sr, 1e-24))
        it = _rsqrt(jnp.maximum(st, 1e-24))
        acc = None
        for kk in range(KD):
          term = jnp.abs(hp[kk] * ih + rp[kk] * ir - tp[kk] * it)
          acc = term if acc is None else acc + term
        svec = jnp.where(lanes_iota == k, jnp.sum(acc), svec)

      out_v[pl.ds(j * CHUNK + g * LANES, LANES)] = svec
      return carry

    lax.fori_loop(0, CHUNK // LANES, group, 0)

  pltpu.sync_copy(out_v, out_hbm.at[pl.ds(base, BPW)])


@functools.partial(
    pl.kernel,
    out_type=jax.ShapeDtypeStruct((B,), jnp.float32),
    mesh=plsc.VectorSubcoreMesh(core_axis_name="c", subcore_axis_name="s"),
    compiler_params=pltpu.CompilerParams(
        needs_layout_passes=False, use_tc_tiling_on_sc=True),
    scratch_types=[
        pltpu.VMEM((NCH, CHUNK), jnp.int32),
        pltpu.VMEM((2 * CHUNK * D,), jnp.float32),
        pltpu.VMEM((2 * CHUNK * D,), jnp.float32),
        pltpu.VMEM((2 * CHUNK, 128), jnp.float32),
        pltpu.VMEM((BPW,), jnp.float32),
        pltpu.SemaphoreType.DMA,
    ],
)
def _score(ridx, rel_pad, scr, out, ri_v, h_v, t_v, r_v, out_v, sem):
  _score_body(ridx, rel_pad, scr, out, ri_v, h_v, t_v, r_v, out_v, sem)


def kernel(data, ent_emb, rel_emb):
  data = data.astype(jnp.int32)
  hidx = data[:, 0]
  ridx = data[:, 1].reshape(NW, NCH, CHUNK)
  tidx = data[:, 2]
  ent3 = ent_emb.T.reshape(1, 8, 8, NE)
  rel_pad = jnp.pad(rel_emb, ((0, 0), (0, 128 - D)))
  scr = _scan(hidx, tidx, ent3)
  return _score(ridx, rel_pad, scr)
